# Initial kernel scaffold; baseline (speedup 1.0000x reference)
import jax
import jax.numpy as jnp
from jax.experimental import pallas as pl


def kernel(x, word_table, pos_table):
    raise NotImplementedError("write your pallas kernel here")


# speedup vs baseline: 1.8805x; 1.8805x over previous
import functools

import jax
import jax.numpy as jnp
from jax import lax
from jax.experimental import pallas as pl
from jax.experimental.pallas import tpu as pltpu
from jax.experimental.pallas import tpu_sc as plsc

_VOCAB = 100000
_EMBED = 128
_BATCH = 1024
_SEQ = 200

_NW = 32
_ROWS = _BATCH * _SEQ
_RPW = _ROWS // _NW
_CH = 128
_NCH = _RPW // _CH

_mesh = plsc.VectorSubcoreMesh(core_axis_name="c", subcore_axis_name="s")


@functools.partial(
    pl.kernel,
    mesh=_mesh,
    out_type=jax.ShapeDtypeStruct((_ROWS, _EMBED), jnp.float32),
    scratch_types=[
        pltpu.VMEM((_NCH, _CH), jnp.int32),
        pltpu.VMEM((2 * _SEQ, _EMBED), jnp.float32),
        pltpu.VMEM((_CH, _EMBED), jnp.float32),
        pltpu.SemaphoreType.DMA,
    ],
)
def _emb_lookup(x_hbm, wt_hbm, pt_hbm, out_hbm, idx_v, pos_v, rows_v, sem):
    wid = lax.axis_index("s") * 2 + lax.axis_index("c")
    base = wid * _RPW

    pltpu.sync_copy(x_hbm.at[wid], idx_v)
    pltpu.sync_copy(pt_hbm.at[pl.ds(0, _SEQ)], pos_v.at[pl.ds(0, _SEQ)])
    pltpu.sync_copy(pt_hbm.at[pl.ds(0, _SEQ)], pos_v.at[pl.ds(_SEQ, _SEQ)])

    def chunk_body(c, carry):
        pltpu.async_copy(wt_hbm.at[idx_v.at[c]], rows_v, sem).wait()

        pos_off = (c * _CH) % _SEQ

        def row_body(r, rcarry):
            p = pos_off + r
            for j in range(_EMBED // 16):
                sl = pl.ds(j * 16, 16)
                rows_v[r, sl] += pos_v[p, sl]
            return rcarry

        lax.fori_loop(0, _CH, row_body, 0)

        pltpu.sync_copy(rows_v, out_hbm.at[pl.ds(base + c * _CH, _CH)])
        return carry

    lax.fori_loop(0, _NCH, chunk_body, 0)


def kernel(x, word_table, pos_table):
    xf = x.reshape(_NW, _NCH, _CH).astype(jnp.int32)
    out = _emb_lookup(xf, word_table, pos_table)
    return out.reshape(_BATCH, _SEQ, _EMBED)


# speedup vs baseline: 2.2855x; 1.2153x over previous
import functools

import jax
import jax.numpy as jnp
from jax import lax
from jax.experimental import pallas as pl
from jax.experimental.pallas import tpu as pltpu
from jax.experimental.pallas import tpu_sc as plsc

_VOCAB = 100000
_EMBED = 128
_BATCH = 1024
_SEQ = 200

_NW = 32
_ROWS = _BATCH * _SEQ
_RPW = _ROWS // _NW
_CH = 128
_NCH = _RPW // _CH
_UNROLL = 4

_mesh = plsc.VectorSubcoreMesh(core_axis_name="c", subcore_axis_name="s")


@functools.partial(
    pl.kernel,
    mesh=_mesh,
    out_type=jax.ShapeDtypeStruct((_ROWS, _EMBED), jnp.float32),
    scratch_types=[
        pltpu.VMEM((_NCH, _CH), jnp.int32),
        pltpu.VMEM((2 * _SEQ, _EMBED), jnp.float32),
        pltpu.VMEM((_CH, _EMBED), jnp.float32),
        pltpu.VMEM((_CH, _EMBED), jnp.float32),
        pltpu.SemaphoreType.DMA,
        pltpu.SemaphoreType.DMA,
        pltpu.SemaphoreType.DMA,
        pltpu.SemaphoreType.DMA,
    ],
)
def _emb_lookup(x_hbm, wt_hbm, pt_hbm, out_hbm, idx_v, pos_v,
                rows0, rows1, gsem0, gsem1, wsem0, wsem1):
    wid = lax.axis_index("s") * 2 + lax.axis_index("c")
    base = wid * _RPW

    pltpu.sync_copy(x_hbm.at[wid], idx_v)
    pltpu.sync_copy(pt_hbm.at[pl.ds(0, _SEQ)], pos_v.at[pl.ds(0, _SEQ)])
    pltpu.sync_copy(pt_hbm.at[pl.ds(0, _SEQ)], pos_v.at[pl.ds(_SEQ, _SEQ)])

    def gather(c, buf, sem):
        pltpu.async_copy(wt_hbm.at[idx_v.at[c]], buf, sem)

    def gather_wait(buf, sem):
        pltpu.make_async_copy(wt_hbm.at[pl.ds(0, _CH)], buf, sem).wait()

    def write(c, buf, sem):
        pltpu.async_copy(buf, out_hbm.at[pl.ds(base + c * _CH, _CH)], sem)

    def write_wait(buf, sem):
        pltpu.make_async_copy(buf, out_hbm.at[pl.ds(base, _CH)], sem).wait()

    def add_pos(c, buf):
        pos_off = (c * _CH) % _SEQ

        def row_body(i, carry):
            r = i * _UNROLL
            for u in range(_UNROLL):
                p = pos_off + r + u
                for j in range(_EMBED // 16):
                    sl = pl.ds(j * 16, 16)
                    buf[r + u, sl] += pos_v[p, sl]
            return carry

        lax.fori_loop(0, _CH // _UNROLL, row_body, 0)

    gather(0, rows0, gsem0)

    def pair_body(pair, carry):
        c0 = 2 * pair
        c1 = c0 + 1

        gather_wait(rows0, gsem0)

        @pl.when(pair > 0)
        def _():
            write_wait(rows1, wsem1)

        gather(c1, rows1, gsem1)
        add_pos(c0, rows0)
        write(c0, rows0, wsem0)

        gather_wait(rows1, gsem1)
        add_pos(c1, rows1)
        write_wait(rows0, wsem0)

        @pl.when(pair < _NCH // 2 - 1)
        def _():
            gather(c0 + 2, rows0, gsem0)

        write(c1, rows1, wsem1)
        return carry

    lax.fori_loop(0, _NCH // 2, pair_body, 0)

    write_wait(rows1, wsem1)


def kernel(x, word_table, pos_table):
    xf = x.reshape(_NW, _NCH, _CH).astype(jnp.int32)
    out = _emb_lookup(xf, word_table, pos_table)
    return out.reshape(_BATCH, _SEQ, _EMBED)


# speedup vs baseline: 6.0372x; 2.6415x over previous
import functools

import jax
import jax.numpy as jnp
import numpy as np
from jax import lax
from jax.experimental import pallas as pl
from jax.experimental.pallas import tpu as pltpu
from jax.experimental.pallas import tpu_sc as plsc

_VOCAB = 100000
_EMBED = 128
_BATCH = 1024
_SEQ = 200

_NW = 32
_ROWS = _BATCH * _SEQ
_CH = 128
_NBLK = _BATCH // _CH
_NCH = _SEQ * _NBLK // _NW
_UNROLL = 2

_OUT_IDX = np.arange(_BATCH)[:, None] * _SEQ + np.arange(_SEQ)[None, :]
_OUT_IDX = np.ascontiguousarray(
    _OUT_IDX.T.reshape(_SEQ, _NBLK, _CH).reshape(_NW, _NCH, _CH)
).astype(np.int32)

_mesh = plsc.VectorSubcoreMesh(core_axis_name="c", subcore_axis_name="s")


@functools.partial(
    pl.kernel,
    mesh=_mesh,
    out_type=jax.ShapeDtypeStruct((_ROWS, _EMBED), jnp.float32),
    scratch_types=[
        pltpu.VMEM((_NCH, _CH), jnp.int32),
        pltpu.VMEM((_NCH, _CH), jnp.int32),
        pltpu.VMEM((_SEQ, _EMBED), jnp.float32),
        pltpu.VMEM((_CH, _EMBED), jnp.float32),
        pltpu.VMEM((_CH, _EMBED), jnp.float32),
        pltpu.SemaphoreType.DMA,
        pltpu.SemaphoreType.DMA,
        pltpu.SemaphoreType.DMA,
        pltpu.SemaphoreType.DMA,
    ],
)
def _emb_lookup(x_hbm, wt_hbm, pt_hbm, oidx_hbm, out_hbm, idx_v, oidx_v,
                pos_v, rows0, rows1, gsem0, gsem1, wsem0, wsem1):
    wid = lax.axis_index("s") * 2 + lax.axis_index("c")

    pltpu.sync_copy(x_hbm.at[wid], idx_v)
    pltpu.sync_copy(oidx_hbm.at[wid], oidx_v)
    pltpu.sync_copy(pt_hbm.at[pl.ds(0, _SEQ)], pos_v)

    def gather(c, buf, sem):
        pltpu.async_copy(wt_hbm.at[idx_v.at[c]], buf, sem)

    def gather_wait(buf, sem):
        pltpu.make_async_copy(wt_hbm.at[pl.ds(0, _CH)], buf, sem).wait()

    def scatter(c, buf, sem):
        pltpu.async_copy(buf, out_hbm.at[oidx_v.at[c]], sem)

    def scatter_wait(c, buf, sem):
        pltpu.make_async_copy(buf, out_hbm.at[oidx_v.at[c]], sem).wait()

    def add_pos(c, buf):
        s_pos = (wid * _NCH + c) // _NBLK
        prow = [pos_v[s_pos, pl.ds(j * 16, 16)] for j in range(_EMBED // 16)]

        def row_body(i, carry):
            r = i * _UNROLL
            for u in range(_UNROLL):
                w = [buf[r + u, pl.ds(j * 16, 16)] for j in range(_EMBED // 16)]
                for j in range(_EMBED // 16):
                    buf[r + u, pl.ds(j * 16, 16)] = w[j] + prow[j]
            return carry

        lax.fori_loop(0, _CH // _UNROLL, row_body, 0)

    gather(0, rows0, gsem0)

    def pair_body(pair, carry):
        c0 = 2 * pair
        c1 = c0 + 1

        gather_wait(rows0, gsem0)

        @pl.when(pair > 0)
        def _():
            scatter_wait(c0 - 1, rows1, wsem1)

        gather(c1, rows1, gsem1)
        add_pos(c0, rows0)
        scatter(c0, rows0, wsem0)

        gather_wait(rows1, gsem1)
        add_pos(c1, rows1)
        scatter_wait(c0, rows0, wsem0)

        @pl.when(pair < _NCH // 2 - 1)
        def _():
            gather(c0 + 2, rows0, gsem0)

        scatter(c1, rows1, wsem1)
        return carry

    lax.fori_loop(0, _NCH // 2, pair_body, 0)

    scatter_wait(_NCH - 1, rows1, wsem1)


def kernel(x, word_table, pos_table):
    xt = x.astype(jnp.int32).T.reshape(_SEQ, _NBLK, _CH).reshape(_NW, _NCH, _CH)
    out = _emb_lookup(xt, word_table, pos_table, jnp.asarray(_OUT_IDX))
    return out.reshape(_BATCH, _SEQ, _EMBED)


# speedup vs baseline: 7.3983x; 1.2255x over previous
import functools

import jax
import jax.numpy as jnp
import numpy as np
from jax import lax
from jax.experimental import pallas as pl
from jax.experimental.pallas import tpu as pltpu
from jax.experimental.pallas import tpu_sc as plsc

_VOCAB = 100000
_EMBED = 128
_BATCH = 1024
_SEQ = 200

_NW = 32
_ROWS = _BATCH * _SEQ
_CH = 128
_NBLK = _BATCH // _CH
_NCH = _SEQ * _NBLK // _NW
_UNROLL = 2

_OUT_IDX = np.arange(_BATCH)[:, None] * _SEQ + np.arange(_SEQ)[None, :]
_OUT_IDX = np.ascontiguousarray(
    _OUT_IDX.T.reshape(_SEQ, _NBLK, _CH).reshape(_NW, _NCH, _CH)
).astype(np.int32)

_mesh = plsc.VectorSubcoreMesh(core_axis_name="c", subcore_axis_name="s")


@functools.partial(
    pl.kernel,
    mesh=_mesh,
    out_type=jax.ShapeDtypeStruct((_ROWS, _EMBED), jnp.float32),
    scratch_types=[
        pltpu.VMEM((_NCH, _CH), jnp.int32),
        pltpu.VMEM((_NCH, _CH), jnp.int32),
        pltpu.VMEM((_SEQ, _EMBED), jnp.float32),
        pltpu.VMEM((_CH, _EMBED), jnp.float32),
        pltpu.VMEM((_CH, _EMBED), jnp.float32),
        pltpu.VMEM((_CH, _EMBED), jnp.float32),
        pltpu.VMEM((_CH, _EMBED), jnp.float32),
        pltpu.SemaphoreType.DMA,
        pltpu.SemaphoreType.DMA,
        pltpu.SemaphoreType.DMA,
        pltpu.SemaphoreType.DMA,
        pltpu.SemaphoreType.DMA,
        pltpu.SemaphoreType.DMA,
        pltpu.SemaphoreType.DMA,
        pltpu.SemaphoreType.DMA,
    ],
)
def _emb_lookup(x_hbm, wt_hbm, pt_hbm, oidx_hbm, out_hbm, idx_v, oidx_v,
                pos_v, rows0, rows1, rows2, rows3,
                gsem0, gsem1, gsem2, gsem3, wsem0, wsem1, wsem2, wsem3):
    wid = lax.axis_index("s") * 2 + lax.axis_index("c")

    pltpu.sync_copy(x_hbm.at[wid], idx_v)
    pltpu.sync_copy(oidx_hbm.at[wid], oidx_v)
    pltpu.sync_copy(pt_hbm.at[pl.ds(0, _SEQ)], pos_v)

    def gather(c, buf, sem):
        pltpu.async_copy(wt_hbm.at[idx_v.at[c]], buf, sem)

    def gather_wait(buf, sem):
        pltpu.make_async_copy(wt_hbm.at[pl.ds(0, _CH)], buf, sem).wait()

    def scatter(c, buf, sem):
        pltpu.async_copy(buf, out_hbm.at[oidx_v.at[c]], sem)

    def scatter_wait(c, buf, sem):
        pltpu.make_async_copy(buf, out_hbm.at[oidx_v.at[c]], sem).wait()

    def add_pos(c, buf):
        s_pos = (wid * _NCH + c) // _NBLK
        prow = [pos_v[s_pos, pl.ds(j * 16, 16)] for j in range(_EMBED // 16)]

        def row_body(i, carry):
            r = i * _UNROLL
            for u in range(_UNROLL):
                w = [buf[r + u, pl.ds(j * 16, 16)] for j in range(_EMBED // 16)]
                for j in range(_EMBED // 16):
                    buf[r + u, pl.ds(j * 16, 16)] = w[j] + prow[j]
            return carry

        lax.fori_loop(0, _CH // _UNROLL, row_body, 0)

    bufs = (rows0, rows1, rows2, rows3)
    gsems = (gsem0, gsem1, gsem2, gsem3)
    wsems = (wsem0, wsem1, wsem2, wsem3)

    def chunk_step(c, b, issue_next):
        gather_wait(bufs[b], gsems[b])
        add_pos(c, bufs[b])
        b2 = (b + 2) % 4
        if issue_next:
            @pl.when(c >= 2)
            def _():
                scatter_wait(c - 2, bufs[b2], wsems[b2])

            gather(c + 2, bufs[b2], gsems[b2])
        else:
            scatter_wait(c - 2, bufs[b2], wsems[b2])
        scatter(c, bufs[b], wsems[b])

    gather(0, rows0, gsem0)
    gather(1, rows1, gsem1)

    def quad_body(q, carry):
        c0 = 4 * q
        for u in range(4):
            chunk_step(c0 + u, u, issue_next=True)
        return carry

    lax.fori_loop(0, (_NCH - 2) // 4, quad_body, 0)

    chunk_step(_NCH - 2, 0, issue_next=False)
    chunk_step(_NCH - 1, 1, issue_next=False)
    scatter_wait(_NCH - 2, rows0, wsem0)
    scatter_wait(_NCH - 1, rows1, wsem1)


def kernel(x, word_table, pos_table):
    xt = x.astype(jnp.int32).T.reshape(_SEQ, _NBLK, _CH).reshape(_NW, _NCH, _CH)
    out = _emb_lookup(xt, word_table, pos_table, jnp.asarray(_OUT_IDX))
    return out.reshape(_BATCH, _SEQ, _EMBED)
